# baseline (speedup 1.0000x reference)
import functools

import jax
import jax.numpy as jnp
from jax import lax
from jax.experimental import pallas as pl
from jax.experimental.pallas import tpu as pltpu
from jax.experimental.pallas import tpu_sc as plsc

B, L, C, D = 4096, 200, 20, 64
CP = 32
CR = 32
NC, NS, LANES = 2, 16, 16
NW = NC * NS
RPW = B // NW
G1 = 128
G2 = L - G1
EV2 = 500000


def _ebody(cids, e2, crows, cidv, cbuf0, cbuf1, sem0, sem1):
    wid = lax.axis_index("s") * NC + lax.axis_index("c")
    base = wid * RPW
    pltpu.sync_copy(cids.at[pl.ds(base * C, RPW * C)], cidv)

    def gather(j, cbuf, sem):
        iv0 = lax.shift_right_logical(cidv[pl.ds(j * C, LANES)], 1)
        iv1 = lax.shift_right_logical(cidv[pl.ds(j * C + 4, LANES)], 1)
        pltpu.async_copy(e2.at[iv0], cbuf.at[pl.ds(0, LANES)], sem)
        pltpu.async_copy(e2.at[iv1], cbuf.at[pl.ds(LANES, LANES)], sem)

    def drain(cbuf, sem):
        pltpu.make_async_copy(e2.at[pl.ds(0, LANES)],
                              cbuf.at[pl.ds(0, LANES)], sem).wait()
        pltpu.make_async_copy(e2.at[pl.ds(0, LANES)],
                              cbuf.at[pl.ds(LANES, LANES)], sem).wait()

    def store(j, cbuf):
        pltpu.sync_copy(cbuf, crows.at[pl.ds((base + j) * CR, CR)])

    gather(0, cbuf0, sem0)
    gather(1, cbuf1, sem1)

    def pair_body(i, _):
        b0 = 2 * i
        drain(cbuf0, sem0)
        store(b0, cbuf0)
        gather(jnp.minimum(b0 + 2, RPW - 1), cbuf0, sem0)
        drain(cbuf1, sem1)
        store(b0 + 1, cbuf1)
        gather(jnp.minimum(b0 + 3, RPW - 1), cbuf1, sem1)
        return 0

    lax.fori_loop(0, RPW // 2, pair_body, 0)
    drain(cbuf0, sem0)
    drain(cbuf1, sem1)


@functools.partial(
    pl.kernel,
    out_type=jax.ShapeDtypeStruct((B * CR, 128), jnp.float32),
    mesh=plsc.VectorSubcoreMesh(core_axis_name="c", subcore_axis_name="s"),
    compiler_params=pltpu.CompilerParams(
        needs_layout_passes=False, use_tc_tiling_on_sc=True),
    scratch_types=[
        pltpu.VMEM((RPW * C,), jnp.int32),
        pltpu.VMEM((CR, 128), jnp.float32),
        pltpu.VMEM((CR, 128), jnp.float32),
        pltpu.SemaphoreType.DMA,
        pltpu.SemaphoreType.DMA,
    ],
)
def _ent_gather_sc(*refs):
    _ebody(*refs)


def _wbody(wids, cids, wtab, crows, wt, bias, out,
           widx, cidv, wrows0, crows0, wrows1, crows1,
           wtv, bv, scoresv, tmat, sem0, sem1):
    wid = lax.axis_index("s") * NC + lax.axis_index("c")
    base = wid * RPW

    pltpu.sync_copy(wids.at[pl.ds(base, RPW)], widx)
    pltpu.sync_copy(cids.at[pl.ds(base * C, RPW * C)], cidv)
    pltpu.sync_copy(wt, wtv)
    pltpu.sync_copy(bias, bv)

    lane = lax.iota(jnp.int32, LANES)

    def issue(j, wrows, crows_v, sem):
        pltpu.async_copy(wtab.at[widx.at[j, pl.ds(0, G1)]],
                         wrows.at[pl.ds(0, G1)], sem)
        pltpu.async_copy(wtab.at[widx.at[j, pl.ds(G1, G2)]],
                         wrows.at[pl.ds(G1, G2)], sem)
        pltpu.async_copy(crows.at[pl.ds((base + j) * CR, CR)],
                         crows_v, sem)

    def drain(wrows, crows_v, sem):
        pltpu.make_async_copy(wtab.at[pl.ds(0, G1)],
                              wrows.at[pl.ds(0, G1)], sem).wait()
        pltpu.make_async_copy(wtab.at[pl.ds(0, G2)],
                              wrows.at[pl.ds(G1, G2)], sem).wait()
        pltpu.make_async_copy(crows.at[pl.ds(0, CR)],
                              crows_v, sem).wait()

    def compute(j, wrows, crows_v):
        one = jnp.ones((LANES,), jnp.float32)
        zf = jnp.zeros((LANES,), jnp.float32)
        z16 = jnp.zeros((LANES,), jnp.int32)
        cntv = zf
        for k in range(L // LANES):
            v = widx[j, pl.ds(k * LANES, LANES)]
            cntv = cntv + jnp.where(v != 0, one, zf)
        vt = widx[j, pl.ds(L - LANES, LANES)]
        cntv = cntv + jnp.where(
            (vt != 0) & (lane >= LANES - (L % LANES)), one, zf)
        tmat[0, pl.ds(0, LANES)] = cntv
        t0 = zf
        t1 = zf
        for k2 in range(0, LANES, 2):
            t0 = t0 + plsc.load_gather(
                tmat, [z16, jnp.full((LANES,), k2, jnp.int32)])
            t1 = t1 + plsc.load_gather(
                tmat, [z16, jnp.full((LANES,), k2 + 1, jnp.int32)])
        inv = one / jnp.maximum(t0 + t1, one)

        def sum_body(li, accs):
            a = list(accs)
            for u in range(8):
                l = li * 8 + u
                for k in range(4):
                    r = wrows[l, pl.ds(k * LANES, LANES)]
                    i = k + 4 * (u % 2)
                    a[i] = a[i] + r
            return tuple(a)

        accs = lax.fori_loop(0, L // 8, sum_body, (zf,) * 8)
        pooled = [(accs[k] + accs[k + 4]) * inv for k in range(4)]

        pacc = [bv[pl.ds(k * LANES, LANES)] for k in range(4)]
        for kin in range(4):
            p = pooled[kin]
            for u in range(LANES):
                s = p[u]
                for k in range(4):
                    pacc[k] = pacc[k] + wtv[kin * LANES + u,
                                            pl.ds(k * LANES, LANES)] * s

        par0 = cidv[pl.ds(j * C, LANES)] & 1
        par1 = cidv[pl.ds(j * C + 4, LANES)] & 1

        def tbuild(c, cc):
            row = c if c < LANES else 12 + c
            par = par0[c] if c < LANES else par1[c - 4]
            t = zf
            for k in range(4):
                h0 = crows_v[row, pl.ds(k * LANES, LANES)]
                h1 = crows_v[row, pl.ds(64 + k * LANES, LANES)]
                t = t + jnp.where(par == 1, h1, h0) * pacc[k]
            tmat[cc, pl.ds(0, LANES)] = t

        def tsum():
            r0 = zf
            r1 = zf
            for k2 in range(0, LANES, 2):
                c0 = plsc.load_gather(
                    tmat, [lane, jnp.full((LANES,), k2, jnp.int32)])
                c1 = plsc.load_gather(
                    tmat, [lane, jnp.full((LANES,), k2 + 1, jnp.int32)])
                r0 = r0 + c0
                r1 = r1 + c1
            return r0 + r1

        for cc in range(LANES):
            tbuild(cc, cc)
        scoresv[j, pl.ds(0, LANES)] = tsum()
        for cc in range(C - LANES):
            tbuild(LANES + cc, cc)
        for cc in range(C - LANES, LANES):
            tmat[cc, pl.ds(0, LANES)] = zf
        scoresv[j, pl.ds(LANES, LANES)] = tsum()

    issue(0, wrows0, crows0, sem0)
    issue(1, wrows1, crows1, sem1)

    def pair_body(i, _):
        b0 = 2 * i
        drain(wrows0, crows0, sem0)
        compute(b0, wrows0, crows0)
        issue(jnp.minimum(b0 + 2, RPW - 1), wrows0, crows0, sem0)
        drain(wrows1, crows1, sem1)
        compute(b0 + 1, wrows1, crows1)
        issue(jnp.minimum(b0 + 3, RPW - 1), wrows1, crows1, sem1)
        return 0

    lax.fori_loop(0, RPW // 2, pair_body, 0)
    drain(wrows0, crows0, sem0)
    drain(wrows1, crows1, sem1)
    pltpu.sync_copy(scoresv, out.at[pl.ds(base, RPW)])


@functools.partial(
    pl.kernel,
    out_type=jax.ShapeDtypeStruct((B, CP), jnp.float32),
    mesh=plsc.VectorSubcoreMesh(core_axis_name="c", subcore_axis_name="s"),
    compiler_params=pltpu.CompilerParams(
        needs_layout_passes=False, use_tc_tiling_on_sc=False),
    scratch_types=[
        pltpu.VMEM((RPW, L), jnp.int32),
        pltpu.VMEM((RPW * C,), jnp.int32),
        pltpu.VMEM((L, D), jnp.float32),
        pltpu.VMEM((CR, 128), jnp.float32),
        pltpu.VMEM((L, D), jnp.float32),
        pltpu.VMEM((CR, 128), jnp.float32),
        pltpu.VMEM((D, D), jnp.float32),
        pltpu.VMEM((D,), jnp.float32),
        pltpu.VMEM((RPW, CP), jnp.float32),
        pltpu.VMEM((LANES, LANES), jnp.float32),
        pltpu.SemaphoreType.DMA,
        pltpu.SemaphoreType.DMA,
    ],
)
def _yamada_sc(*refs):
    _wbody(*refs)


def kernel(word_ids, cand_ids, word_table, ent_table, W, b):
    wids = word_ids.astype(jnp.int32)
    cids = cand_ids.astype(jnp.int32).reshape(B * C)
    e2 = ent_table.reshape(EV2, 128)
    wt = jnp.transpose(W)
    crows = _ent_gather_sc(cids, e2)
    return _yamada_sc(wids, cids, word_table, crows, wt, b)[:, :C]


# speedup vs baseline: 1.0707x; 1.0707x over previous
import functools

import jax
import jax.numpy as jnp
from jax import lax
from jax.experimental import pallas as pl
from jax.experimental.pallas import tpu as pltpu
from jax.experimental.pallas import tpu_sc as plsc

B, L, C, D = 4096, 200, 20, 64
CP = 32
CR = 32
NC, NS, LANES = 2, 16, 16
NW = NC * NS
RPW = B // NW
G1 = 128
G2 = L - G1
EV2 = 500000


def _ebody(cids, e2, crows, cidv, cbuf0, cbuf1, sem0, sem1):
    wid = lax.axis_index("s") * NC + lax.axis_index("c")
    base = wid * RPW
    pltpu.sync_copy(cids.at[pl.ds(base * C, RPW * C)], cidv)

    def gather(j, cbuf, sem):
        iv0 = cidv[pl.ds(j * C, LANES)]
        iv1 = cidv[pl.ds(j * C + 4, LANES)]
        pltpu.async_copy(e2.at[iv0], cbuf.at[pl.ds(0, LANES)], sem)
        pltpu.async_copy(e2.at[iv1], cbuf.at[pl.ds(LANES, LANES)], sem)

    def drain(cbuf, sem):
        pltpu.make_async_copy(e2.at[pl.ds(0, LANES)],
                              cbuf.at[pl.ds(0, LANES)], sem).wait()
        pltpu.make_async_copy(e2.at[pl.ds(0, LANES)],
                              cbuf.at[pl.ds(LANES, LANES)], sem).wait()

    def store(j, cbuf):
        pltpu.sync_copy(cbuf, crows.at[pl.ds((base + j) * CR, CR)])

    gather(0, cbuf0, sem0)
    gather(1, cbuf1, sem1)

    def pair_body(i, _):
        b0 = 2 * i
        drain(cbuf0, sem0)
        store(b0, cbuf0)
        gather(jnp.minimum(b0 + 2, RPW - 1), cbuf0, sem0)
        drain(cbuf1, sem1)
        store(b0 + 1, cbuf1)
        gather(jnp.minimum(b0 + 3, RPW - 1), cbuf1, sem1)
        return 0

    lax.fori_loop(0, RPW // 2, pair_body, 0)
    drain(cbuf0, sem0)
    drain(cbuf1, sem1)


@functools.partial(
    pl.kernel,
    out_type=jax.ShapeDtypeStruct((B * CR, 128), jnp.float32),
    mesh=plsc.VectorSubcoreMesh(core_axis_name="c", subcore_axis_name="s"),
    compiler_params=pltpu.CompilerParams(
        needs_layout_passes=False, use_tc_tiling_on_sc=True),
    scratch_types=[
        pltpu.VMEM((RPW * C,), jnp.int32),
        pltpu.VMEM((CR, 128), jnp.float32),
        pltpu.VMEM((CR, 128), jnp.float32),
        pltpu.SemaphoreType.DMA,
        pltpu.SemaphoreType.DMA,
    ],
)
def _ent_gather_sc(*refs):
    _ebody(*refs)


def _wbody(wids, wtab, crows, wt, bias, out,
           widx, wrows0, crows0, wrows1, crows1,
           wtv, bv, scoresv, tmat, sem0, sem1):
    wid = lax.axis_index("s") * NC + lax.axis_index("c")
    base = wid * RPW

    pltpu.sync_copy(wids.at[pl.ds(base, RPW)], widx)
    pltpu.sync_copy(wt, wtv)
    pltpu.sync_copy(bias, bv)

    lane = lax.iota(jnp.int32, LANES)

    def issue(j, wrows, crows_v, sem):
        pltpu.async_copy(wtab.at[widx.at[j, pl.ds(0, G1)]],
                         wrows.at[pl.ds(0, G1)], sem)
        pltpu.async_copy(wtab.at[widx.at[j, pl.ds(G1, G2)]],
                         wrows.at[pl.ds(G1, G2)], sem)
        pltpu.async_copy(crows.at[pl.ds((base + j) * CR, CR)],
                         crows_v, sem)

    def drain(wrows, crows_v, sem):
        pltpu.make_async_copy(wtab.at[pl.ds(0, G1)],
                              wrows.at[pl.ds(0, G1)], sem).wait()
        pltpu.make_async_copy(wtab.at[pl.ds(0, G2)],
                              wrows.at[pl.ds(G1, G2)], sem).wait()
        pltpu.make_async_copy(crows.at[pl.ds(0, CR)],
                              crows_v, sem).wait()

    def compute(j, wrows, crows_v):
        one = jnp.ones((LANES,), jnp.float32)
        zf = jnp.zeros((LANES,), jnp.float32)
        z16 = jnp.zeros((LANES,), jnp.int32)
        cntv = zf
        for k in range(L // LANES):
            v = widx[j, pl.ds(k * LANES, LANES)]
            cntv = cntv + jnp.where(v != 0, one, zf)
        vt = widx[j, pl.ds(L - LANES, LANES)]
        cntv = cntv + jnp.where(
            (vt != 0) & (lane >= LANES - (L % LANES)), one, zf)
        tmat[0, pl.ds(0, LANES)] = cntv
        t0 = zf
        t1 = zf
        for k2 in range(0, LANES, 2):
            t0 = t0 + plsc.load_gather(
                tmat, [z16, jnp.full((LANES,), k2, jnp.int32)])
            t1 = t1 + plsc.load_gather(
                tmat, [z16, jnp.full((LANES,), k2 + 1, jnp.int32)])
        inv = one / jnp.maximum(t0 + t1, one)

        def sum_body(li, accs):
            a = list(accs)
            for u in range(8):
                l = li * 8 + u
                for k in range(4):
                    r = wrows[l, pl.ds(k * LANES, LANES)]
                    i = k + 4 * (u % 2)
                    a[i] = a[i] + r
            return tuple(a)

        accs = lax.fori_loop(0, L // 8, sum_body, (zf,) * 8)
        pooled = [(accs[k] + accs[k + 4]) * inv for k in range(4)]

        pacc = [bv[pl.ds(k * LANES, LANES)] for k in range(4)]
        for kin in range(4):
            p = pooled[kin]
            for u in range(LANES):
                s = p[u]
                for k in range(4):
                    pacc[k] = pacc[k] + wtv[kin * LANES + u,
                                            pl.ds(k * LANES, LANES)] * s

        def tbuild(c, cc):
            row = c if c < LANES else 12 + c
            t = crows_v[row, pl.ds(0, LANES)] * pacc[0]
            for k in range(1, 4):
                t = t + crows_v[row, pl.ds(k * LANES, LANES)] * pacc[k]
            tmat[cc, pl.ds(0, LANES)] = t

        def tsum():
            r0 = zf
            r1 = zf
            for k2 in range(0, LANES, 2):
                c0 = plsc.load_gather(
                    tmat, [lane, jnp.full((LANES,), k2, jnp.int32)])
                c1 = plsc.load_gather(
                    tmat, [lane, jnp.full((LANES,), k2 + 1, jnp.int32)])
                r0 = r0 + c0
                r1 = r1 + c1
            return r0 + r1

        for cc in range(LANES):
            tbuild(cc, cc)
        scoresv[j, pl.ds(0, LANES)] = tsum()
        for cc in range(C - LANES):
            tbuild(LANES + cc, cc)
        for cc in range(C - LANES, LANES):
            tmat[cc, pl.ds(0, LANES)] = zf
        scoresv[j, pl.ds(LANES, LANES)] = tsum()

    issue(0, wrows0, crows0, sem0)
    issue(1, wrows1, crows1, sem1)

    def pair_body(i, _):
        b0 = 2 * i
        drain(wrows0, crows0, sem0)
        compute(b0, wrows0, crows0)
        issue(jnp.minimum(b0 + 2, RPW - 1), wrows0, crows0, sem0)
        drain(wrows1, crows1, sem1)
        compute(b0 + 1, wrows1, crows1)
        issue(jnp.minimum(b0 + 3, RPW - 1), wrows1, crows1, sem1)
        return 0

    lax.fori_loop(0, RPW // 2, pair_body, 0)
    drain(wrows0, crows0, sem0)
    drain(wrows1, crows1, sem1)
    pltpu.sync_copy(scoresv, out.at[pl.ds(base, RPW)])


@functools.partial(
    pl.kernel,
    out_type=jax.ShapeDtypeStruct((B, CP), jnp.float32),
    mesh=plsc.VectorSubcoreMesh(core_axis_name="c", subcore_axis_name="s"),
    compiler_params=pltpu.CompilerParams(
        needs_layout_passes=False, use_tc_tiling_on_sc=False),
    scratch_types=[
        pltpu.VMEM((RPW, L), jnp.int32),
        pltpu.VMEM((L, D), jnp.float32),
        pltpu.VMEM((CR, 128), jnp.float32),
        pltpu.VMEM((L, D), jnp.float32),
        pltpu.VMEM((CR, 128), jnp.float32),
        pltpu.VMEM((D, D), jnp.float32),
        pltpu.VMEM((D,), jnp.float32),
        pltpu.VMEM((RPW, CP), jnp.float32),
        pltpu.VMEM((LANES, LANES), jnp.float32),
        pltpu.SemaphoreType.DMA,
        pltpu.SemaphoreType.DMA,
    ],
)
def _yamada_sc(*refs):
    _wbody(*refs)


def kernel(word_ids, cand_ids, word_table, ent_table, W, b):
    wids = word_ids.astype(jnp.int32)
    cids = cand_ids.astype(jnp.int32).reshape(B * C)
    e2 = jnp.pad(ent_table, ((0, 0), (0, 64)))
    wt = jnp.transpose(W)
    crows = _ent_gather_sc(cids, e2)
    return _yamada_sc(wids, word_table, crows, wt, b)[:, :C]


# speedup vs baseline: 1.1376x; 1.0625x over previous
import functools

import jax
import jax.numpy as jnp
from jax import lax
from jax.experimental import pallas as pl
from jax.experimental.pallas import tpu as pltpu
from jax.experimental.pallas import tpu_sc as plsc

B, L, C, D = 4096, 200, 20, 64
CP = 32
CR = 32
NC, NS, LANES = 2, 16, 16
NW = NC * NS
RPW = B // NW
G1 = 128
G2 = L - G1


def _w1body(wids, wtab, wt, bias, out,
            widx, wrows0, wrows1, wtv, bv, projv, tmat, sem0, sem1):
    wid = lax.axis_index("s") * NC + lax.axis_index("c")
    base = wid * RPW

    pltpu.sync_copy(wids.at[pl.ds(base, RPW)], widx)
    pltpu.sync_copy(wt, wtv)
    pltpu.sync_copy(bias, bv)

    lane = lax.iota(jnp.int32, LANES)

    def issue(j, wrows, sem):
        pltpu.async_copy(wtab.at[widx.at[j, pl.ds(0, G1)]],
                         wrows.at[pl.ds(0, G1)], sem)
        pltpu.async_copy(wtab.at[widx.at[j, pl.ds(G1, G2)]],
                         wrows.at[pl.ds(G1, G2)], sem)

    def drain(wrows, sem):
        pltpu.make_async_copy(wtab.at[pl.ds(0, G1)],
                              wrows.at[pl.ds(0, G1)], sem).wait()
        pltpu.make_async_copy(wtab.at[pl.ds(0, G2)],
                              wrows.at[pl.ds(G1, G2)], sem).wait()

    def compute(j, wrows):
        one = jnp.ones((LANES,), jnp.float32)
        zf = jnp.zeros((LANES,), jnp.float32)
        z16 = jnp.zeros((LANES,), jnp.int32)
        cntv = zf
        for k in range(L // LANES):
            v = widx[j, pl.ds(k * LANES, LANES)]
            cntv = cntv + jnp.where(v != 0, one, zf)
        vt = widx[j, pl.ds(L - LANES, LANES)]
        cntv = cntv + jnp.where(
            (vt != 0) & (lane >= LANES - (L % LANES)), one, zf)
        tmat[0, pl.ds(0, LANES)] = cntv
        t0 = zf
        t1 = zf
        for k2 in range(0, LANES, 2):
            t0 = t0 + plsc.load_gather(
                tmat, [z16, jnp.full((LANES,), k2, jnp.int32)])
            t1 = t1 + plsc.load_gather(
                tmat, [z16, jnp.full((LANES,), k2 + 1, jnp.int32)])
        inv = one / jnp.maximum(t0 + t1, one)

        def sum_body(li, accs):
            a = list(accs)
            for u in range(8):
                l = li * 8 + u
                for k in range(4):
                    r = wrows[l, pl.ds(k * LANES, LANES)]
                    i = k + 4 * (u % 2)
                    a[i] = a[i] + r
            return tuple(a)

        accs = lax.fori_loop(0, L // 8, sum_body, (zf,) * 8)
        pooled = [(accs[k] + accs[k + 4]) * inv for k in range(4)]

        pacc = [bv[pl.ds(k * LANES, LANES)] for k in range(4)]
        for kin in range(4):
            p = pooled[kin]
            for u in range(LANES):
                s = p[u]
                for k in range(4):
                    pacc[k] = pacc[k] + wtv[kin * LANES + u,
                                            pl.ds(k * LANES, LANES)] * s
        for k in range(4):
            projv[pl.ds(j * D + k * LANES, LANES)] = pacc[k]

    issue(0, wrows0, sem0)
    issue(1, wrows1, sem1)

    def pair_body(i, _):
        b0 = 2 * i
        drain(wrows0, sem0)
        compute(b0, wrows0)
        issue(jnp.minimum(b0 + 2, RPW - 1), wrows0, sem0)
        drain(wrows1, sem1)
        compute(b0 + 1, wrows1)
        issue(jnp.minimum(b0 + 3, RPW - 1), wrows1, sem1)
        return 0

    lax.fori_loop(0, RPW // 2, pair_body, 0)
    drain(wrows0, sem0)
    drain(wrows1, sem1)
    pltpu.sync_copy(projv, out.at[pl.ds(base * D, RPW * D)])


@functools.partial(
    pl.kernel,
    out_type=jax.ShapeDtypeStruct((B * D,), jnp.float32),
    mesh=plsc.VectorSubcoreMesh(core_axis_name="c", subcore_axis_name="s"),
    compiler_params=pltpu.CompilerParams(
        needs_layout_passes=False, use_tc_tiling_on_sc=False),
    scratch_types=[
        pltpu.VMEM((RPW, L), jnp.int32),
        pltpu.VMEM((L, D), jnp.float32),
        pltpu.VMEM((L, D), jnp.float32),
        pltpu.VMEM((D, D), jnp.float32),
        pltpu.VMEM((D,), jnp.float32),
        pltpu.VMEM((RPW * D,), jnp.float32),
        pltpu.VMEM((LANES, LANES), jnp.float32),
        pltpu.SemaphoreType.DMA,
        pltpu.SemaphoreType.DMA,
    ],
)
def _word_proj_sc(*refs):
    _w1body(*refs)


def _ebody(cids, e2, crows, cidv, cbuf0, cc0, cbuf1, cc1, sem0, sem1):
    wid = lax.axis_index("s") * NC + lax.axis_index("c")
    base = wid * RPW
    pltpu.sync_copy(cids.at[pl.ds(base * C, RPW * C)], cidv)

    def gather(j, cbuf, sem):
        iv0 = cidv[pl.ds(j * C, LANES)]
        iv1 = cidv[pl.ds(j * C + 4, LANES)]
        pltpu.async_copy(e2.at[iv0], cbuf.at[pl.ds(0, LANES)], sem)
        pltpu.async_copy(e2.at[iv1], cbuf.at[pl.ds(LANES, LANES)], sem)

    def drain(cbuf, sem):
        pltpu.make_async_copy(e2.at[pl.ds(0, LANES)],
                              cbuf.at[pl.ds(0, LANES)], sem).wait()
        pltpu.make_async_copy(e2.at[pl.ds(0, LANES)],
                              cbuf.at[pl.ds(LANES, LANES)], sem).wait()

    def compact_store(j, cbuf, cc):
        for c in range(C):
            row = c if c < LANES else 12 + c
            for k in range(4):
                cc[pl.ds(c * D + k * LANES, LANES)] = (
                    cbuf[row, pl.ds(k * LANES, LANES)])
        pltpu.sync_copy(cc, crows.at[pl.ds((base + j) * C * D, C * D)])

    gather(0, cbuf0, sem0)
    gather(1, cbuf1, sem1)

    def pair_body(i, _):
        b0 = 2 * i
        drain(cbuf0, sem0)
        compact_store(b0, cbuf0, cc0)
        gather(jnp.minimum(b0 + 2, RPW - 1), cbuf0, sem0)
        drain(cbuf1, sem1)
        compact_store(b0 + 1, cbuf1, cc1)
        gather(jnp.minimum(b0 + 3, RPW - 1), cbuf1, sem1)
        return 0

    lax.fori_loop(0, RPW // 2, pair_body, 0)
    drain(cbuf0, sem0)
    drain(cbuf1, sem1)


@functools.partial(
    pl.kernel,
    out_type=jax.ShapeDtypeStruct((B * C * D,), jnp.float32),
    mesh=plsc.VectorSubcoreMesh(core_axis_name="c", subcore_axis_name="s"),
    compiler_params=pltpu.CompilerParams(
        needs_layout_passes=False, use_tc_tiling_on_sc=True),
    scratch_types=[
        pltpu.VMEM((RPW * C,), jnp.int32),
        pltpu.VMEM((CR, 128), jnp.float32),
        pltpu.VMEM((C * D,), jnp.float32),
        pltpu.VMEM((CR, 128), jnp.float32),
        pltpu.VMEM((C * D,), jnp.float32),
        pltpu.SemaphoreType.DMA,
        pltpu.SemaphoreType.DMA,
    ],
)
def _ent_gather_sc(*refs):
    _ebody(*refs)


def _sbody(proj, crows, out, projv, cr0, cr1, scoresv, tmat, sem0, sem1):
    wid = lax.axis_index("s") * NC + lax.axis_index("c")
    base = wid * RPW
    pltpu.sync_copy(proj.at[pl.ds(base * D, RPW * D)], projv)

    zf = jnp.zeros((LANES,), jnp.float32)
    lane = lax.iota(jnp.int32, LANES)

    def issue(j, cr, sem):
        pltpu.async_copy(crows.at[pl.ds((base + j) * C * D, C * D)], cr, sem)

    def drain(cr, sem):
        pltpu.make_async_copy(crows.at[pl.ds(0, C * D)], cr, sem).wait()

    def compute(j, cr):
        pacc = [projv[pl.ds(j * D + k * LANES, LANES)] for k in range(4)]

        def tbuild(c, cc):
            t = cr[pl.ds(c * D, LANES)] * pacc[0]
            for k in range(1, 4):
                t = t + cr[pl.ds(c * D + k * LANES, LANES)] * pacc[k]
            tmat[cc, pl.ds(0, LANES)] = t

        def tsum():
            r0 = zf
            r1 = zf
            for k2 in range(0, LANES, 2):
                c0 = plsc.load_gather(
                    tmat, [lane, jnp.full((LANES,), k2, jnp.int32)])
                c1 = plsc.load_gather(
                    tmat, [lane, jnp.full((LANES,), k2 + 1, jnp.int32)])
                r0 = r0 + c0
                r1 = r1 + c1
            return r0 + r1

        for cc in range(LANES):
            tbuild(cc, cc)
        scoresv[j, pl.ds(0, LANES)] = tsum()
        for cc in range(C - LANES):
            tbuild(LANES + cc, cc)
        for cc in range(C - LANES, LANES):
            tmat[cc, pl.ds(0, LANES)] = zf
        scoresv[j, pl.ds(LANES, LANES)] = tsum()

    issue(0, cr0, sem0)
    issue(1, cr1, sem1)

    def pair_body(i, _):
        b0 = 2 * i
        drain(cr0, sem0)
        compute(b0, cr0)
        issue(jnp.minimum(b0 + 2, RPW - 1), cr0, sem0)
        drain(cr1, sem1)
        compute(b0 + 1, cr1)
        issue(jnp.minimum(b0 + 3, RPW - 1), cr1, sem1)
        return 0

    lax.fori_loop(0, RPW // 2, pair_body, 0)
    drain(cr0, sem0)
    drain(cr1, sem1)
    pltpu.sync_copy(scoresv, out.at[pl.ds(base, RPW)])


@functools.partial(
    pl.kernel,
    out_type=jax.ShapeDtypeStruct((B, CP), jnp.float32),
    mesh=plsc.VectorSubcoreMesh(core_axis_name="c", subcore_axis_name="s"),
    compiler_params=pltpu.CompilerParams(
        needs_layout_passes=False, use_tc_tiling_on_sc=False),
    scratch_types=[
        pltpu.VMEM((RPW * D,), jnp.float32),
        pltpu.VMEM((C * D,), jnp.float32),
        pltpu.VMEM((C * D,), jnp.float32),
        pltpu.VMEM((RPW, CP), jnp.float32),
        pltpu.VMEM((LANES, LANES), jnp.float32),
        pltpu.SemaphoreType.DMA,
        pltpu.SemaphoreType.DMA,
    ],
)
def _scores_sc(*refs):
    _sbody(*refs)


def kernel(word_ids, cand_ids, word_table, ent_table, W, b):
    wids = word_ids.astype(jnp.int32)
    cids = cand_ids.astype(jnp.int32).reshape(B * C)
    e2 = jnp.pad(ent_table, ((0, 0), (0, 64)))
    wt = jnp.transpose(W)
    proj = _word_proj_sc(wids, word_table, wt, b)
    crows = _ent_gather_sc(cids, e2)
    scores = _scores_sc(proj, crows)
    return scores[:, :C]
